# Initial kernel scaffold; baseline (speedup 1.0000x reference)
import jax
import jax.numpy as jnp
from jax.experimental import pallas as pl


def kernel(query, keys):
    raise NotImplementedError("write your pallas kernel here")


# speedup vs baseline: 24.3775x; 24.3775x over previous
import functools

import jax
import jax.numpy as jnp
from jax.experimental import pallas as pl
from jax.experimental.pallas import tpu as pltpu

N = 8192
M = 10000
D = 64
R = 128
G = N // R
BS = 8
HW = 1024


def _norm_body(q_ref, qr_ref):
    x = q_ref[0]
    ss = jnp.sum(x * x, axis=0, keepdims=True)
    inv = 1.0 / jnp.maximum(jnp.sqrt(ss), 1e-12)
    qr_ref[0] = (x * inv).T


def _keys_body(keys_ref, kt_ref, k2_ref, ks_ref):
    k = keys_ref[...]
    kt = k.T
    kt_ref[...] = kt
    k2_ref[...] = jnp.sum(kt * kt, axis=0, keepdims=True)
    ks_ref[...] = jnp.sum(kt, axis=0, keepdims=True)


def _passA_body(qr_ref, kt_ref, keys_ref, k2_ref, ks_ref,
                sm_ref, cm_ref, gi_ref, cmax_ref, csum_ref, gl_ref, sl_ref):
    i = pl.program_id(0)
    q = qr_ref[...]
    s = jnp.dot(q, kt_ref[...], preferred_element_type=jnp.float32)

    m1 = jnp.max(s, axis=1, keepdims=True)
    p = jnp.exp(s - m1)
    rsum = jnp.sum(p, axis=1, keepdims=True)
    smv = p / rsum
    sm_ref[...] = smv
    cm_ref[...] = jnp.dot(smv, keys_ref[...], preferred_element_type=jnp.float32)

    iota = jax.lax.broadcasted_iota(jnp.int32, (R, M), 1)
    gi = jnp.min(jnp.where(s == m1, iota, M), axis=1, keepdims=True)
    oh1 = iota == gi
    masked = jnp.where(oh1, -jnp.inf, s)
    m2 = jnp.max(masked, axis=1, keepdims=True)
    i2 = jnp.min(jnp.where(masked == m2, iota, M), axis=1, keepdims=True)
    oh2 = iota == i2
    gi_ref[...] = gi

    k2 = k2_ref[...]
    ks = ks_ref[...]
    zero = jnp.float32(0.0)
    k2g = jnp.sum(jnp.where(oh1, k2, zero), axis=1, keepdims=True)
    ksg = jnp.sum(jnp.where(oh1, ks, zero), axis=1, keepdims=True)
    k2n = jnp.sum(jnp.where(oh2, k2, zero), axis=1, keepdims=True)
    ksn = jnp.sum(jnp.where(oh2, ks, zero), axis=1, keepdims=True)

    q2 = jnp.sum(q * q, axis=1, keepdims=True)
    qs = jnp.sum(q, axis=1, keepdims=True)
    c = jnp.float32(1e-6)
    glr = q2 - 2.0 * m1 + k2g
    dp = jnp.sqrt(glr + 2.0 * c * (qs - ksg) + D * c * c)
    dn = jnp.sqrt(q2 - 2.0 * m2 + k2n + 2.0 * c * (qs - ksn) + D * c * c)
    gl_part = jnp.sum(glr)
    sl_part = jnp.sum(jnp.maximum(dp - dn + 1.0, 0.0))

    tmax = jnp.max(s, axis=0, keepdims=True)

    @pl.when(i == 0)
    def _init():
        cmax_ref[...] = tmax
        csum_ref[...] = jnp.sum(jnp.exp(s - tmax), axis=0, keepdims=True)
        gl_ref[0, 0] = gl_part
        sl_ref[0, 0] = sl_part

    @pl.when(i > 0)
    def _acc():
        old = cmax_ref[...]
        new = jnp.maximum(old, tmax)
        csum_ref[...] = (csum_ref[...] * jnp.exp(old - new)
                         + jnp.sum(jnp.exp(s - new), axis=0, keepdims=True))
        cmax_ref[...] = new
        gl_ref[0, 0] += gl_part
        sl_ref[0, 0] += sl_part

    @pl.when(i == G - 1)
    def _fin():
        gl_ref[0, 0] = gl_ref[0, 0] / (N * D)
        sl_ref[0, 0] = sl_ref[0, 0] / N


def _passB_body(qr_ref, cm_ref, kt_ref, gi_ref, cmax_ref, csum_ref,
                sq_ref, uq_ref, um_ref, acc_ref):
    i = pl.program_id(0)
    q = qr_ref[...]
    s = jnp.dot(q, kt_ref[...], preferred_element_type=jnp.float32)
    e = jnp.exp(s - cmax_ref[...])
    sq_ref[...] = e * (1.0 / csum_ref[...])

    iota = jax.lax.broadcasted_iota(jnp.int32, (R, M), 1)
    mw = jnp.where(iota == gi_ref[...], e, 0.0)
    qt = q.T
    contrib = jnp.dot(qt, mw, preferred_element_type=jnp.float32)

    @pl.when(i == 0)
    def _init():
        acc_ref[...] = contrib

    @pl.when(i > 0)
    def _acc():
        acc_ref[...] += contrib

    uq_ref[0, 0:D, :] = qt
    uq_ref[0, D:2 * D, :] = cm_ref[...].T

    @pl.when(i == G - 1)
    def _fin():
        upd = acc_ref[...] + kt_ref[...]
        nrm = jnp.sqrt(jnp.sum(upd * upd, axis=0, keepdims=True))
        um_ref[...] = (upd / jnp.maximum(nrm, 1e-12)).T


@functools.partial(jax.jit, static_argnames=())
def kernel(query, keys):
    f32 = jnp.float32
    qv = query.reshape(BS, D, HW)

    qr3 = pl.pallas_call(
        _norm_body,
        grid=(BS,),
        in_specs=[pl.BlockSpec((1, D, HW), lambda b: (b, 0, 0))],
        out_specs=pl.BlockSpec((1, HW, D), lambda b: (b, 0, 0)),
        out_shape=jax.ShapeDtypeStruct((BS, HW, D), f32),
    )(qv)
    qr = qr3.reshape(N, D)

    kt, k2, ks = pl.pallas_call(
        _keys_body,
        grid=(1,),
        in_specs=[pl.BlockSpec((M, D), lambda _: (0, 0))],
        out_specs=[
            pl.BlockSpec((D, M), lambda _: (0, 0)),
            pl.BlockSpec((1, M), lambda _: (0, 0)),
            pl.BlockSpec((1, M), lambda _: (0, 0)),
        ],
        out_shape=[
            jax.ShapeDtypeStruct((D, M), f32),
            jax.ShapeDtypeStruct((1, M), f32),
            jax.ShapeDtypeStruct((1, M), f32),
        ],
    )(keys)

    sm, cm, gi, cmax, csum, gl, sl = pl.pallas_call(
        _passA_body,
        grid=(G,),
        in_specs=[
            pl.BlockSpec((R, D), lambda i: (i, 0)),
            pl.BlockSpec((D, M), lambda i: (0, 0)),
            pl.BlockSpec((M, D), lambda i: (0, 0)),
            pl.BlockSpec((1, M), lambda i: (0, 0)),
            pl.BlockSpec((1, M), lambda i: (0, 0)),
        ],
        out_specs=[
            pl.BlockSpec((R, M), lambda i: (i, 0)),
            pl.BlockSpec((R, D), lambda i: (i, 0)),
            pl.BlockSpec((R, 1), lambda i: (i, 0)),
            pl.BlockSpec((1, M), lambda i: (0, 0)),
            pl.BlockSpec((1, M), lambda i: (0, 0)),
            pl.BlockSpec(memory_space=pltpu.SMEM),
            pl.BlockSpec(memory_space=pltpu.SMEM),
        ],
        out_shape=[
            jax.ShapeDtypeStruct((N, M), f32),
            jax.ShapeDtypeStruct((N, D), f32),
            jax.ShapeDtypeStruct((N, 1), jnp.int32),
            jax.ShapeDtypeStruct((1, M), f32),
            jax.ShapeDtypeStruct((1, M), f32),
            jax.ShapeDtypeStruct((1, 1), f32),
            jax.ShapeDtypeStruct((1, 1), f32),
        ],
    )(qr, kt, keys, k2, ks)

    sub = HW // R
    sq, uq, um = pl.pallas_call(
        _passB_body,
        grid=(G,),
        in_specs=[
            pl.BlockSpec((R, D), lambda i: (i, 0)),
            pl.BlockSpec((R, D), lambda i: (i, 0)),
            pl.BlockSpec((D, M), lambda i: (0, 0)),
            pl.BlockSpec((R, 1), lambda i: (i, 0)),
            pl.BlockSpec((1, M), lambda i: (0, 0)),
            pl.BlockSpec((1, M), lambda i: (0, 0)),
        ],
        out_specs=[
            pl.BlockSpec((R, M), lambda i: (i, 0)),
            pl.BlockSpec((1, 2 * D, R), lambda i: (i // sub, 0, i % sub)),
            pl.BlockSpec((M, D), lambda i: (0, 0)),
        ],
        out_shape=[
            jax.ShapeDtypeStruct((N, M), f32),
            jax.ShapeDtypeStruct((BS, 2 * D, HW), f32),
            jax.ShapeDtypeStruct((M, D), f32),
        ],
        scratch_shapes=[pltpu.VMEM((D, M), f32)],
    )(qr, cm, kt, gi, cmax, csum)

    updated_query = uq.reshape(BS, 2 * D, 32, 32)
    return (updated_query, um, sq, sm, gl[0, 0], sl[0, 0])


# speedup vs baseline: 27.0331x; 1.1089x over previous
import functools

import jax
import jax.numpy as jnp
from jax.experimental import pallas as pl
from jax.experimental.pallas import tpu as pltpu

N = 8192
M = 10000
D = 64
R = 128
G = N // R
BS = 8
HW = 1024


def _norm_body(q_ref, qr_ref):
    x = q_ref[0]
    ss = jnp.sum(x * x, axis=0, keepdims=True)
    inv = 1.0 / jnp.maximum(jnp.sqrt(ss), 1e-12)
    qr_ref[0] = (x * inv).T


def _keys_body(keys_ref, kt_ref, k2_ref, ks_ref):
    k = keys_ref[...]
    kt = k.T
    kt_ref[...] = kt
    k2_ref[...] = jnp.sum(kt * kt, axis=0, keepdims=True)
    ks_ref[...] = jnp.sum(kt, axis=0, keepdims=True)


def _passA_body(qr_ref, kt_ref, keys_ref, k2_ref, ks_ref,
                sm_ref, cm_ref, m1_ref, cmax_ref, csum_ref, gl_ref, sl_ref):
    i = pl.program_id(0)
    q = qr_ref[...]
    s = jnp.dot(q, kt_ref[...], preferred_element_type=jnp.float32)

    m1 = jnp.max(s, axis=1, keepdims=True)
    p = jnp.exp(s - m1)
    rsum = jnp.sum(p, axis=1, keepdims=True)
    smv = p * (1.0 / rsum)
    sm_ref[...] = smv
    cm_ref[...] = jnp.dot(smv, keys_ref[...], preferred_element_type=jnp.float32)
    m1_ref[...] = m1

    ninf = jnp.float32(-jnp.inf)
    eq = s == m1
    masked = jnp.where(eq, ninf, s)
    m2 = jnp.max(masked, axis=1, keepdims=True)
    eq2 = masked == m2
    k2 = k2_ref[...]
    ks = ks_ref[...]
    k2g = jnp.max(jnp.where(eq, k2, ninf), axis=1, keepdims=True)
    ksg = jnp.max(jnp.where(eq, ks, ninf), axis=1, keepdims=True)
    k2n = jnp.max(jnp.where(eq2, k2, ninf), axis=1, keepdims=True)
    ksn = jnp.max(jnp.where(eq2, ks, ninf), axis=1, keepdims=True)

    q2 = jnp.sum(q * q, axis=1, keepdims=True)
    qs = jnp.sum(q, axis=1, keepdims=True)
    c = jnp.float32(1e-6)
    glr = q2 - 2.0 * m1 + k2g
    dp = jnp.sqrt(glr + 2.0 * c * (qs - ksg) + D * c * c)
    dn = jnp.sqrt(q2 - 2.0 * m2 + k2n + 2.0 * c * (qs - ksn) + D * c * c)
    gl_part = jnp.sum(glr)
    sl_part = jnp.sum(jnp.maximum(dp - dn + 1.0, 0.0))

    tmax = jnp.max(s, axis=0, keepdims=True)
    gm = jnp.max(m1)
    w = jnp.exp(m1.T - gm)
    t = jnp.dot(w, p, preferred_element_type=jnp.float32)

    @pl.when(i == 0)
    def _init():
        cmax_ref[...] = tmax
        csum_ref[...] = t * jnp.exp(gm - tmax)
        gl_ref[0, 0] = gl_part
        sl_ref[0, 0] = sl_part

    @pl.when(i > 0)
    def _acc():
        old = cmax_ref[...]
        nm = jnp.maximum(old, tmax)
        csum_ref[...] = (csum_ref[...] * jnp.exp(old - nm)
                         + t * jnp.exp(gm - nm))
        cmax_ref[...] = nm
        gl_ref[0, 0] += gl_part
        sl_ref[0, 0] += sl_part

    @pl.when(i == G - 1)
    def _fin():
        gl_ref[0, 0] = gl_ref[0, 0] / (N * D)
        sl_ref[0, 0] = sl_ref[0, 0] / N


def _passB_body(qr_ref, cm_ref, kt_ref, m1_ref, cmax_ref, csum_ref,
                sq_ref, uq_ref, um_ref, acc_ref):
    i = pl.program_id(0)
    q = qr_ref[...]
    s = jnp.dot(q, kt_ref[...], preferred_element_type=jnp.float32)
    e = jnp.exp(s - cmax_ref[...])
    sq_ref[...] = e * (1.0 / csum_ref[...])

    mw = jnp.where(s == m1_ref[...], e, 0.0)
    qt = q.T
    contrib = jnp.dot(qt, mw, preferred_element_type=jnp.float32)

    @pl.when(i == 0)
    def _init():
        acc_ref[...] = contrib

    @pl.when(i > 0)
    def _acc():
        acc_ref[...] += contrib

    uq_ref[0, 0:D, :] = qt
    uq_ref[0, D:2 * D, :] = cm_ref[...].T

    @pl.when(i == G - 1)
    def _fin():
        upd = acc_ref[...] + kt_ref[...]
        nrm = jnp.sqrt(jnp.sum(upd * upd, axis=0, keepdims=True))
        um_ref[...] = (upd / jnp.maximum(nrm, 1e-12)).T


@functools.partial(jax.jit, static_argnames=())
def kernel(query, keys):
    f32 = jnp.float32
    qv = query.reshape(BS, D, HW)

    qr3 = pl.pallas_call(
        _norm_body,
        grid=(BS,),
        in_specs=[pl.BlockSpec((1, D, HW), lambda b: (b, 0, 0))],
        out_specs=pl.BlockSpec((1, HW, D), lambda b: (b, 0, 0)),
        out_shape=jax.ShapeDtypeStruct((BS, HW, D), f32),
    )(qv)
    qr = qr3.reshape(N, D)

    kt, k2, ks = pl.pallas_call(
        _keys_body,
        grid=(1,),
        in_specs=[pl.BlockSpec((M, D), lambda _: (0, 0))],
        out_specs=[
            pl.BlockSpec((D, M), lambda _: (0, 0)),
            pl.BlockSpec((1, M), lambda _: (0, 0)),
            pl.BlockSpec((1, M), lambda _: (0, 0)),
        ],
        out_shape=[
            jax.ShapeDtypeStruct((D, M), f32),
            jax.ShapeDtypeStruct((1, M), f32),
            jax.ShapeDtypeStruct((1, M), f32),
        ],
    )(keys)

    sm, cm, m1o, cmax, csum, gl, sl = pl.pallas_call(
        _passA_body,
        grid=(G,),
        in_specs=[
            pl.BlockSpec((R, D), lambda i: (i, 0)),
            pl.BlockSpec((D, M), lambda i: (0, 0)),
            pl.BlockSpec((M, D), lambda i: (0, 0)),
            pl.BlockSpec((1, M), lambda i: (0, 0)),
            pl.BlockSpec((1, M), lambda i: (0, 0)),
        ],
        out_specs=[
            pl.BlockSpec((R, M), lambda i: (i, 0)),
            pl.BlockSpec((R, D), lambda i: (i, 0)),
            pl.BlockSpec((R, 1), lambda i: (i, 0)),
            pl.BlockSpec((1, M), lambda i: (0, 0)),
            pl.BlockSpec((1, M), lambda i: (0, 0)),
            pl.BlockSpec(memory_space=pltpu.SMEM),
            pl.BlockSpec(memory_space=pltpu.SMEM),
        ],
        out_shape=[
            jax.ShapeDtypeStruct((N, M), f32),
            jax.ShapeDtypeStruct((N, D), f32),
            jax.ShapeDtypeStruct((N, 1), f32),
            jax.ShapeDtypeStruct((1, M), f32),
            jax.ShapeDtypeStruct((1, M), f32),
            jax.ShapeDtypeStruct((1, 1), f32),
            jax.ShapeDtypeStruct((1, 1), f32),
        ],
    )(qr, kt, keys, k2, ks)

    sub = HW // R
    sq, uq, um = pl.pallas_call(
        _passB_body,
        grid=(G,),
        in_specs=[
            pl.BlockSpec((R, D), lambda i: (i, 0)),
            pl.BlockSpec((R, D), lambda i: (i, 0)),
            pl.BlockSpec((D, M), lambda i: (0, 0)),
            pl.BlockSpec((R, 1), lambda i: (i, 0)),
            pl.BlockSpec((1, M), lambda i: (0, 0)),
            pl.BlockSpec((1, M), lambda i: (0, 0)),
        ],
        out_specs=[
            pl.BlockSpec((R, M), lambda i: (i, 0)),
            pl.BlockSpec((1, 2 * D, R), lambda i: (i // sub, 0, i % sub)),
            pl.BlockSpec((M, D), lambda i: (0, 0)),
        ],
        out_shape=[
            jax.ShapeDtypeStruct((N, M), f32),
            jax.ShapeDtypeStruct((BS, 2 * D, HW), f32),
            jax.ShapeDtypeStruct((M, D), f32),
        ],
        scratch_shapes=[pltpu.VMEM((D, M), f32)],
    )(qr, cm, kt, m1o, cmax, csum)

    updated_query = uq.reshape(BS, 2 * D, 32, 32)
    return (updated_query, um, sq, sm, gl[0, 0], sl[0, 0])


# speedup vs baseline: 29.6926x; 1.0984x over previous
import functools

import jax
import jax.numpy as jnp
from jax.experimental import pallas as pl
from jax.experimental.pallas import tpu as pltpu

N = 8192
M = 10000
D = 64
R = 128
G = N // R
BS = 8
HW = 1024


def _norm_body(q_ref, qr_ref):
    x = q_ref[0]
    ss = jnp.sum(x * x, axis=0, keepdims=True)
    inv = 1.0 / jnp.maximum(jnp.sqrt(ss), 1e-12)
    qr_ref[0] = (x * inv).T


def _keys_body(keys_ref, kt_ref, k2_ref, b_ref, ebc_ref):
    k = keys_ref[...]
    kt = k.T
    kt_ref[...] = kt
    k2 = jnp.sum(kt * kt, axis=0, keepdims=True)
    k2_ref[...] = k2
    b = jnp.sqrt(k2)
    b_ref[...] = b
    ebc_ref[...] = jnp.exp(jnp.max(b) - b)


def _passA_body(qr_ref, kt_ref, keys_ref, b_ref, ebc_ref,
                sm_ref, cm_ref, csum_ref):
    i = pl.program_id(0)
    q = qr_ref[...]
    s = jnp.dot(q, kt_ref[...], preferred_element_type=jnp.float32)

    c = jnp.max(b_ref[...])
    p = jnp.exp(s - c)
    rsum = jnp.sum(p, axis=1, keepdims=True)
    smv = p * (1.0 / rsum)
    sm_ref[...] = smv
    cm_ref[...] = jnp.dot(smv, keys_ref[...], preferred_element_type=jnp.float32)

    ones = jnp.ones((1, R), jnp.float32)
    colp = jnp.dot(ones, p, preferred_element_type=jnp.float32)
    contrib = colp * ebc_ref[...]

    @pl.when(i == 0)
    def _init():
        csum_ref[...] = contrib

    @pl.when(i > 0)
    def _acc():
        csum_ref[...] += contrib


def _passB_body(qr_ref, cm_ref, kt_ref, k2_ref, b_ref, csum_ref,
                sq_ref, uq_ref, um_ref, gl_ref, sl_ref, acc_ref, cem_ref):
    i = pl.program_id(0)
    q = qr_ref[...]
    s = jnp.dot(q, kt_ref[...], preferred_element_type=jnp.float32)
    e = jnp.exp(s - b_ref[...])
    sq_ref[...] = e * (1.0 / csum_ref[...])

    ninf = jnp.float32(-jnp.inf)
    m1 = jnp.max(s, axis=1, keepdims=True)
    eq = s == m1
    mw = jnp.where(eq, e, 0.0)
    qt = q.T
    contrib = jnp.dot(qt, mw, preferred_element_type=jnp.float32)
    cem = jnp.max(e, axis=0, keepdims=True)

    masked = jnp.where(eq, ninf, s)
    m2 = jnp.max(masked, axis=1, keepdims=True)
    eq2 = masked == m2
    k2 = k2_ref[...]
    k2g = jnp.max(jnp.where(eq, k2, ninf), axis=1, keepdims=True)
    k2n = jnp.max(jnp.where(eq2, k2, ninf), axis=1, keepdims=True)

    q2 = jnp.sum(q * q, axis=1, keepdims=True)
    glr = q2 - 2.0 * m1 + k2g
    dp = jnp.sqrt(glr)
    dn = jnp.sqrt(q2 - 2.0 * m2 + k2n)
    gl_part = jnp.sum(glr)
    sl_part = jnp.sum(jnp.maximum(dp - dn + 1.0, 0.0))

    uq_ref[0, 0:D, :] = qt
    uq_ref[0, D:2 * D, :] = cm_ref[...].T

    @pl.when(i == 0)
    def _init():
        acc_ref[...] = contrib
        cem_ref[...] = cem
        gl_ref[0, 0] = gl_part
        sl_ref[0, 0] = sl_part

    @pl.when(i > 0)
    def _acc():
        acc_ref[...] += contrib
        cem_ref[...] = jnp.maximum(cem_ref[...], cem)
        gl_ref[0, 0] += gl_part
        sl_ref[0, 0] += sl_part

    @pl.when(i == G - 1)
    def _fin():
        gl_ref[0, 0] = gl_ref[0, 0] / (N * D)
        sl_ref[0, 0] = sl_ref[0, 0] / N
        upd = acc_ref[...] * (1.0 / cem_ref[...]) + kt_ref[...]
        nrm = jnp.sqrt(jnp.sum(upd * upd, axis=0, keepdims=True))
        um_ref[...] = (upd / jnp.maximum(nrm, 1e-12)).T


@functools.partial(jax.jit, static_argnames=())
def kernel(query, keys):
    f32 = jnp.float32
    qv = query.reshape(BS, D, HW)

    qr3 = pl.pallas_call(
        _norm_body,
        grid=(BS,),
        in_specs=[pl.BlockSpec((1, D, HW), lambda b: (b, 0, 0))],
        out_specs=pl.BlockSpec((1, HW, D), lambda b: (b, 0, 0)),
        out_shape=jax.ShapeDtypeStruct((BS, HW, D), f32),
    )(qv)
    qr = qr3.reshape(N, D)

    kt, k2, b, ebc = pl.pallas_call(
        _keys_body,
        grid=(1,),
        in_specs=[pl.BlockSpec((M, D), lambda _: (0, 0))],
        out_specs=[
            pl.BlockSpec((D, M), lambda _: (0, 0)),
            pl.BlockSpec((1, M), lambda _: (0, 0)),
            pl.BlockSpec((1, M), lambda _: (0, 0)),
            pl.BlockSpec((1, M), lambda _: (0, 0)),
        ],
        out_shape=[
            jax.ShapeDtypeStruct((D, M), f32),
            jax.ShapeDtypeStruct((1, M), f32),
            jax.ShapeDtypeStruct((1, M), f32),
            jax.ShapeDtypeStruct((1, M), f32),
        ],
    )(keys)

    sm, cm, csum = pl.pallas_call(
        _passA_body,
        grid=(G,),
        in_specs=[
            pl.BlockSpec((R, D), lambda i: (i, 0)),
            pl.BlockSpec((D, M), lambda i: (0, 0)),
            pl.BlockSpec((M, D), lambda i: (0, 0)),
            pl.BlockSpec((1, M), lambda i: (0, 0)),
            pl.BlockSpec((1, M), lambda i: (0, 0)),
        ],
        out_specs=[
            pl.BlockSpec((R, M), lambda i: (i, 0)),
            pl.BlockSpec((R, D), lambda i: (i, 0)),
            pl.BlockSpec((1, M), lambda i: (0, 0)),
        ],
        out_shape=[
            jax.ShapeDtypeStruct((N, M), f32),
            jax.ShapeDtypeStruct((N, D), f32),
            jax.ShapeDtypeStruct((1, M), f32),
        ],
    )(qr, kt, keys, b, ebc)

    sub = HW // R
    sq, uq, um, gl, sl = pl.pallas_call(
        _passB_body,
        grid=(G,),
        in_specs=[
            pl.BlockSpec((R, D), lambda i: (i, 0)),
            pl.BlockSpec((R, D), lambda i: (i, 0)),
            pl.BlockSpec((D, M), lambda i: (0, 0)),
            pl.BlockSpec((1, M), lambda i: (0, 0)),
            pl.BlockSpec((1, M), lambda i: (0, 0)),
            pl.BlockSpec((1, M), lambda i: (0, 0)),
        ],
        out_specs=[
            pl.BlockSpec((R, M), lambda i: (i, 0)),
            pl.BlockSpec((1, 2 * D, R), lambda i: (i // sub, 0, i % sub)),
            pl.BlockSpec((M, D), lambda i: (0, 0)),
            pl.BlockSpec(memory_space=pltpu.SMEM),
            pl.BlockSpec(memory_space=pltpu.SMEM),
        ],
        out_shape=[
            jax.ShapeDtypeStruct((N, M), f32),
            jax.ShapeDtypeStruct((BS, 2 * D, HW), f32),
            jax.ShapeDtypeStruct((M, D), f32),
            jax.ShapeDtypeStruct((1, 1), f32),
            jax.ShapeDtypeStruct((1, 1), f32),
        ],
        scratch_shapes=[pltpu.VMEM((D, M), f32), pltpu.VMEM((1, M), f32)],
    )(qr, cm, kt, k2, b, csum)

    updated_query = uq.reshape(BS, 2 * D, 32, 32)
    return (updated_query, um, sq, sm, gl[0, 0], sl[0, 0])


# speedup vs baseline: 29.8778x; 1.0062x over previous
import functools

import jax
import jax.numpy as jnp
from jax.experimental import pallas as pl
from jax.experimental.pallas import tpu as pltpu

N = 8192
M = 10000
D = 64
RA = 256
GA = N // RA
RB = 256
GB = N // RB
BS = 8
HW = 1024


def _norm_body(q_ref, qr_ref):
    x = q_ref[0]
    ss = jnp.sum(x * x, axis=0, keepdims=True)
    inv = 1.0 / jnp.maximum(jnp.sqrt(ss), 1e-12)
    qr_ref[0] = (x * inv).T


def _keys_body(keys_ref, kt_ref, k2_ref, b_ref, ebc_ref):
    k = keys_ref[...]
    kt = k.T
    kt_ref[...] = kt
    k2 = jnp.sum(kt * kt, axis=0, keepdims=True)
    k2_ref[...] = k2
    b = jnp.sqrt(k2)
    b_ref[...] = b
    ebc_ref[...] = jnp.exp(jnp.max(b) - b)


def _passA_body(qr_ref, kt_ref, keys_ref, b_ref, ebc_ref,
                sm_ref, cm_ref, csum_ref):
    i = pl.program_id(0)
    q = qr_ref[...]
    s = jnp.dot(q, kt_ref[...], preferred_element_type=jnp.float32)

    c = jnp.max(b_ref[...])
    p = jnp.exp(s - c)
    rsum = jnp.sum(p, axis=1, keepdims=True)
    smv = p * (1.0 / rsum)
    sm_ref[...] = smv
    cm_ref[...] = jnp.dot(smv, keys_ref[...], preferred_element_type=jnp.float32)

    ones = jnp.ones((1, RA), jnp.float32)
    colp = jnp.dot(ones, p, preferred_element_type=jnp.float32)
    contrib = colp * ebc_ref[...]

    @pl.when(i == 0)
    def _init():
        csum_ref[...] = contrib

    @pl.when(i > 0)
    def _acc():
        csum_ref[...] += contrib


def _passB_body(qr_ref, cm_ref, kt_ref, k2_ref, b_ref, csum_ref,
                sq_ref, uq_ref, um_ref, gl_ref, sl_ref, acc_ref, cem_ref):
    i = pl.program_id(0)
    q = qr_ref[...]
    s = jnp.dot(q, kt_ref[...], preferred_element_type=jnp.float32)
    e = jnp.exp(s - b_ref[...])
    sq_ref[...] = e * (1.0 / csum_ref[...])

    ninf = jnp.float32(-jnp.inf)
    m1 = jnp.max(s, axis=1, keepdims=True)
    eq = s == m1
    mw = jnp.where(eq, e, 0.0)
    qt = q.T
    contrib = jnp.dot(qt, mw, preferred_element_type=jnp.float32)
    cem = jnp.max(e, axis=0, keepdims=True)

    masked = jnp.where(eq, ninf, s)
    m2 = jnp.max(masked, axis=1, keepdims=True)
    eq2 = masked == m2
    k2 = k2_ref[...]
    k2g = jnp.max(jnp.where(eq, k2, ninf), axis=1, keepdims=True)
    k2n = jnp.max(jnp.where(eq2, k2, ninf), axis=1, keepdims=True)

    q2 = jnp.sum(q * q, axis=1, keepdims=True)
    glr = q2 - 2.0 * m1 + k2g
    dp = jnp.sqrt(glr)
    dn = jnp.sqrt(q2 - 2.0 * m2 + k2n)
    gl_part = jnp.sum(glr)
    sl_part = jnp.sum(jnp.maximum(dp - dn + 1.0, 0.0))

    uq_ref[0, 0:D, :] = qt
    uq_ref[0, D:2 * D, :] = cm_ref[...].T

    @pl.when(i == 0)
    def _init():
        acc_ref[...] = contrib
        cem_ref[...] = cem
        gl_ref[0, 0] = gl_part
        sl_ref[0, 0] = sl_part

    @pl.when(i > 0)
    def _acc():
        acc_ref[...] += contrib
        cem_ref[...] = jnp.maximum(cem_ref[...], cem)
        gl_ref[0, 0] += gl_part
        sl_ref[0, 0] += sl_part

    @pl.when(i == GB - 1)
    def _fin():
        gl_ref[0, 0] = gl_ref[0, 0] / (N * D)
        sl_ref[0, 0] = sl_ref[0, 0] / N
        upd = acc_ref[...] * (1.0 / cem_ref[...]) + kt_ref[...]
        nrm = jnp.sqrt(jnp.sum(upd * upd, axis=0, keepdims=True))
        um_ref[...] = (upd / jnp.maximum(nrm, 1e-12)).T


@functools.partial(jax.jit, static_argnames=())
def kernel(query, keys):
    f32 = jnp.float32
    qv = query.reshape(BS, D, HW)

    qr3 = pl.pallas_call(
        _norm_body,
        grid=(BS,),
        in_specs=[pl.BlockSpec((1, D, HW), lambda b: (b, 0, 0))],
        out_specs=pl.BlockSpec((1, HW, D), lambda b: (b, 0, 0)),
        out_shape=jax.ShapeDtypeStruct((BS, HW, D), f32),
    )(qv)
    qr = qr3.reshape(N, D)

    kt, k2, b, ebc = pl.pallas_call(
        _keys_body,
        grid=(1,),
        in_specs=[pl.BlockSpec((M, D), lambda _: (0, 0))],
        out_specs=[
            pl.BlockSpec((D, M), lambda _: (0, 0)),
            pl.BlockSpec((1, M), lambda _: (0, 0)),
            pl.BlockSpec((1, M), lambda _: (0, 0)),
            pl.BlockSpec((1, M), lambda _: (0, 0)),
        ],
        out_shape=[
            jax.ShapeDtypeStruct((D, M), f32),
            jax.ShapeDtypeStruct((1, M), f32),
            jax.ShapeDtypeStruct((1, M), f32),
            jax.ShapeDtypeStruct((1, M), f32),
        ],
    )(keys)

    sm, cm, csum = pl.pallas_call(
        _passA_body,
        grid=(GA,),
        in_specs=[
            pl.BlockSpec((RA, D), lambda i: (i, 0)),
            pl.BlockSpec((D, M), lambda i: (0, 0)),
            pl.BlockSpec((M, D), lambda i: (0, 0)),
            pl.BlockSpec((1, M), lambda i: (0, 0)),
            pl.BlockSpec((1, M), lambda i: (0, 0)),
        ],
        out_specs=[
            pl.BlockSpec((RA, M), lambda i: (i, 0)),
            pl.BlockSpec((RA, D), lambda i: (i, 0)),
            pl.BlockSpec((1, M), lambda i: (0, 0)),
        ],
        out_shape=[
            jax.ShapeDtypeStruct((N, M), f32),
            jax.ShapeDtypeStruct((N, D), f32),
            jax.ShapeDtypeStruct((1, M), f32),
        ],
    )(qr, kt, keys, b, ebc)

    sub = HW // RB
    sq, uq, um, gl, sl = pl.pallas_call(
        _passB_body,
        grid=(GB,),
        in_specs=[
            pl.BlockSpec((RB, D), lambda i: (i, 0)),
            pl.BlockSpec((RB, D), lambda i: (i, 0)),
            pl.BlockSpec((D, M), lambda i: (0, 0)),
            pl.BlockSpec((1, M), lambda i: (0, 0)),
            pl.BlockSpec((1, M), lambda i: (0, 0)),
            pl.BlockSpec((1, M), lambda i: (0, 0)),
        ],
        out_specs=[
            pl.BlockSpec((RB, M), lambda i: (i, 0)),
            pl.BlockSpec((1, 2 * D, RB), lambda i: (i // sub, 0, i % sub)),
            pl.BlockSpec((M, D), lambda i: (0, 0)),
            pl.BlockSpec(memory_space=pltpu.SMEM),
            pl.BlockSpec(memory_space=pltpu.SMEM),
        ],
        out_shape=[
            jax.ShapeDtypeStruct((N, M), f32),
            jax.ShapeDtypeStruct((BS, 2 * D, HW), f32),
            jax.ShapeDtypeStruct((M, D), f32),
            jax.ShapeDtypeStruct((1, 1), f32),
            jax.ShapeDtypeStruct((1, 1), f32),
        ],
        scratch_shapes=[pltpu.VMEM((D, M), f32), pltpu.VMEM((1, M), f32)],
    )(qr, cm, kt, k2, b, csum)

    updated_query = uq.reshape(BS, 2 * D, 32, 32)
    return (updated_query, um, sq, sm, gl[0, 0], sl[0, 0])


# speedup vs baseline: 30.4340x; 1.0186x over previous
import functools

import jax
import jax.numpy as jnp
from jax.experimental import pallas as pl
from jax.experimental.pallas import tpu as pltpu

N = 8192
M = 10000
D = 64
RA = 128
GA = N // RA
RB = 256
GB = N // RB
BS = 8
HW = 1024


def _norm_body(q_ref, qr_ref):
    x = q_ref[0]
    ss = jnp.sum(x * x, axis=0, keepdims=True)
    inv = 1.0 / jnp.maximum(jnp.sqrt(ss), 1e-12)
    qr_ref[0] = (x * inv).T


def _keys_body(keys_ref, kt_ref, k2_ref, b_ref, ebc_ref):
    k = keys_ref[...]
    kt = k.T
    kt_ref[...] = kt
    k2 = jnp.sum(kt * kt, axis=0, keepdims=True)
    k2_ref[...] = k2
    b = jnp.sqrt(k2)
    b_ref[...] = b
    ebc_ref[...] = jnp.exp(jnp.max(b) - b)


def _passA_body(qr_ref, kt_ref, keys_ref, b_ref, ebc_ref,
                sm_ref, cm_ref, csum_ref):
    i = pl.program_id(0)
    q = qr_ref[...]
    s = jnp.dot(q, kt_ref[...], preferred_element_type=jnp.float32)

    c = jnp.max(b_ref[...])
    p = jnp.exp(s - c)
    rsum = jnp.sum(p, axis=1, keepdims=True)
    smv = p * (1.0 / rsum)
    sm_ref[...] = smv
    cm_ref[...] = jnp.dot(smv, keys_ref[...], preferred_element_type=jnp.float32)

    ones = jnp.ones((1, RA), jnp.float32)
    colp = jnp.dot(ones, p, preferred_element_type=jnp.float32)
    contrib = colp * ebc_ref[...]

    @pl.when(i == 0)
    def _init():
        csum_ref[...] = contrib

    @pl.when(i > 0)
    def _acc():
        csum_ref[...] += contrib


def _passB_body(qr_ref, cm_ref, kt_ref, k2_ref, b_ref, csum_ref,
                sq_ref, uq_ref, um_ref, gl_ref, sl_ref, acc_ref, cem_ref):
    i = pl.program_id(0)
    q = qr_ref[...]
    s = jnp.dot(q, kt_ref[...], preferred_element_type=jnp.float32)
    e = jnp.exp(s - b_ref[...])
    sq_ref[...] = e * (1.0 / csum_ref[...])

    ninf = jnp.float32(-jnp.inf)
    m1 = jnp.max(s, axis=1, keepdims=True)
    eq = s == m1
    mw = jnp.where(eq, e, 0.0)
    qt = q.T
    contrib = jnp.dot(qt, mw, preferred_element_type=jnp.float32)
    cem = jnp.max(e, axis=0, keepdims=True)

    m2 = jnp.max(jnp.where(eq, ninf, s), axis=1, keepdims=True)
    eq2 = s == m2
    k2 = k2_ref[...]
    k2g = jnp.max(jnp.where(eq, k2, ninf), axis=1, keepdims=True)
    k2n = jnp.max(jnp.where(eq2, k2, ninf), axis=1, keepdims=True)

    q2 = jnp.sum(q * q, axis=1, keepdims=True)
    glr = q2 - 2.0 * m1 + k2g
    dp = jnp.sqrt(glr)
    dn = jnp.sqrt(q2 - 2.0 * m2 + k2n)
    gl_part = jnp.sum(glr)
    sl_part = jnp.sum(jnp.maximum(dp - dn + 1.0, 0.0))

    uq_ref[0, 0:D, :] = qt
    uq_ref[0, D:2 * D, :] = cm_ref[...].T

    @pl.when(i == 0)
    def _init():
        acc_ref[...] = contrib
        cem_ref[...] = cem
        gl_ref[0, 0] = gl_part
        sl_ref[0, 0] = sl_part

    @pl.when(i > 0)
    def _acc():
        acc_ref[...] += contrib
        cem_ref[...] = jnp.maximum(cem_ref[...], cem)
        gl_ref[0, 0] += gl_part
        sl_ref[0, 0] += sl_part

    @pl.when(i == GB - 1)
    def _fin():
        gl_ref[0, 0] = gl_ref[0, 0] / (N * D)
        sl_ref[0, 0] = sl_ref[0, 0] / N
        upd = acc_ref[...] * (1.0 / cem_ref[...]) + kt_ref[...]
        nrm = jnp.sqrt(jnp.sum(upd * upd, axis=0, keepdims=True))
        um_ref[...] = (upd / jnp.maximum(nrm, 1e-12)).T


@functools.partial(jax.jit, static_argnames=())
def kernel(query, keys):
    f32 = jnp.float32
    qv = query.reshape(BS, D, HW)

    qr3 = pl.pallas_call(
        _norm_body,
        grid=(BS,),
        in_specs=[pl.BlockSpec((1, D, HW), lambda b: (b, 0, 0))],
        out_specs=pl.BlockSpec((1, HW, D), lambda b: (b, 0, 0)),
        out_shape=jax.ShapeDtypeStruct((BS, HW, D), f32),
    )(qv)
    qr = qr3.reshape(N, D)

    kt, k2, b, ebc = pl.pallas_call(
        _keys_body,
        grid=(1,),
        in_specs=[pl.BlockSpec((M, D), lambda _: (0, 0))],
        out_specs=[
            pl.BlockSpec((D, M), lambda _: (0, 0)),
            pl.BlockSpec((1, M), lambda _: (0, 0)),
            pl.BlockSpec((1, M), lambda _: (0, 0)),
            pl.BlockSpec((1, M), lambda _: (0, 0)),
        ],
        out_shape=[
            jax.ShapeDtypeStruct((D, M), f32),
            jax.ShapeDtypeStruct((1, M), f32),
            jax.ShapeDtypeStruct((1, M), f32),
            jax.ShapeDtypeStruct((1, M), f32),
        ],
    )(keys)

    sm, cm, csum = pl.pallas_call(
        _passA_body,
        grid=(GA,),
        in_specs=[
            pl.BlockSpec((RA, D), lambda i: (i, 0)),
            pl.BlockSpec((D, M), lambda i: (0, 0)),
            pl.BlockSpec((M, D), lambda i: (0, 0)),
            pl.BlockSpec((1, M), lambda i: (0, 0)),
            pl.BlockSpec((1, M), lambda i: (0, 0)),
        ],
        out_specs=[
            pl.BlockSpec((RA, M), lambda i: (i, 0)),
            pl.BlockSpec((RA, D), lambda i: (i, 0)),
            pl.BlockSpec((1, M), lambda i: (0, 0)),
        ],
        out_shape=[
            jax.ShapeDtypeStruct((N, M), f32),
            jax.ShapeDtypeStruct((N, D), f32),
            jax.ShapeDtypeStruct((1, M), f32),
        ],
    )(qr, kt, keys, b, ebc)

    sub = HW // RB
    sq, uq, um, gl, sl = pl.pallas_call(
        _passB_body,
        grid=(GB,),
        in_specs=[
            pl.BlockSpec((RB, D), lambda i: (i, 0)),
            pl.BlockSpec((RB, D), lambda i: (i, 0)),
            pl.BlockSpec((D, M), lambda i: (0, 0)),
            pl.BlockSpec((1, M), lambda i: (0, 0)),
            pl.BlockSpec((1, M), lambda i: (0, 0)),
            pl.BlockSpec((1, M), lambda i: (0, 0)),
        ],
        out_specs=[
            pl.BlockSpec((RB, M), lambda i: (i, 0)),
            pl.BlockSpec((1, 2 * D, RB), lambda i: (i // sub, 0, i % sub)),
            pl.BlockSpec((M, D), lambda i: (0, 0)),
            pl.BlockSpec(memory_space=pltpu.SMEM),
            pl.BlockSpec(memory_space=pltpu.SMEM),
        ],
        out_shape=[
            jax.ShapeDtypeStruct((N, M), f32),
            jax.ShapeDtypeStruct((BS, 2 * D, HW), f32),
            jax.ShapeDtypeStruct((M, D), f32),
            jax.ShapeDtypeStruct((1, 1), f32),
            jax.ShapeDtypeStruct((1, 1), f32),
        ],
        scratch_shapes=[pltpu.VMEM((D, M), f32), pltpu.VMEM((1, M), f32)],
    )(qr, cm, kt, k2, b, csum)

    updated_query = uq.reshape(BS, 2 * D, 32, 32)
    return (updated_query, um, sq, sm, gl[0, 0], sl[0, 0])


# speedup vs baseline: 30.4513x; 1.0006x over previous
import functools

import jax
import jax.numpy as jnp
from jax.experimental import pallas as pl
from jax.experimental.pallas import tpu as pltpu

N = 8192
M = 10000
D = 64
RA = 128
GA = N // RA
RB = 256
GB = N // RB
BS = 8
HW = 1024


def _norm_body(q_ref, qr_ref):
    x = q_ref[0]
    ss = jnp.sum(x * x, axis=0, keepdims=True)
    inv = 1.0 / jnp.maximum(jnp.sqrt(ss), 1e-12)
    qr_ref[0] = (x * inv).T


def _keys_body(keys_ref, kt_ref, k2c_ref, b_ref, ebc_ref):
    k = keys_ref[...]
    kt = k.T
    kt_ref[...] = kt
    k2 = jnp.sum(kt * kt, axis=0, keepdims=True)
    k2c_ref[...] = k2
    b = jnp.sqrt(k2)
    b_ref[...] = b
    ebc_ref[...] = jnp.exp(jnp.max(b) - b)


def _passA_body(qr_ref, kt_ref, keys_ref, b_ref, ebc_ref,
                sm_ref, cm_ref, csum_ref):
    i = pl.program_id(0)
    q = qr_ref[...]
    s = jnp.dot(q, kt_ref[...], preferred_element_type=jnp.float32)

    c = jnp.max(b_ref[...])
    p = jnp.exp(s - c)
    rsum = jnp.sum(p, axis=1, keepdims=True)
    smv = p * (1.0 / rsum)
    sm_ref[...] = smv
    cm_ref[...] = jnp.dot(smv, keys_ref[...], preferred_element_type=jnp.float32)

    ones = jnp.ones((1, RA), jnp.float32)
    colp = jnp.dot(ones, p, preferred_element_type=jnp.float32)
    contrib = colp * ebc_ref[...]

    @pl.when(i == 0)
    def _init():
        csum_ref[...] = contrib

    @pl.when(i > 0)
    def _acc():
        csum_ref[...] += contrib


def _passB_body(qr_ref, cm_ref, kt_ref, k2c_ref, b_ref, csum_ref,
                sq_ref, uq_ref, um_ref, gl_ref, sl_ref, acc_ref, cem_ref):
    i = pl.program_id(0)
    q = qr_ref[...]
    s = jnp.dot(q, kt_ref[...], preferred_element_type=jnp.float32)
    e = jnp.exp(s - b_ref[...])
    sq_ref[...] = e * (1.0 / csum_ref[...])

    ninf = jnp.float32(-jnp.inf)
    m1 = jnp.max(s, axis=1, keepdims=True)
    eq = s == m1
    mw = jnp.where(eq, e, 0.0)
    qt = q.T
    contrib = jnp.dot(qt, mw, preferred_element_type=jnp.float32)
    cem = jnp.max(e, axis=0, keepdims=True)

    m2 = jnp.max(jnp.where(eq, ninf, s), axis=1, keepdims=True)
    eq2 = s == m2
    k2 = k2c_ref[...]
    k2g = jnp.max(jnp.where(eq, k2, ninf), axis=1, keepdims=True)
    k2n = jnp.max(jnp.where(eq2, k2, ninf), axis=1, keepdims=True)

    q2 = jnp.sum(q * q, axis=1, keepdims=True)
    glr = q2 - 2.0 * m1 + k2g
    dp = jnp.sqrt(glr)
    dn = jnp.sqrt(q2 - 2.0 * m2 + k2n)
    gl_part = jnp.sum(glr)
    sl_part = jnp.sum(jnp.maximum(dp - dn + 1.0, 0.0))

    uq_ref[0, 0:D, :] = qt
    uq_ref[0, D:2 * D, :] = cm_ref[...].T

    @pl.when(i == 0)
    def _init():
        acc_ref[...] = contrib
        cem_ref[...] = cem
        gl_ref[0, 0] = gl_part
        sl_ref[0, 0] = sl_part

    @pl.when(i > 0)
    def _acc():
        acc_ref[...] += contrib
        cem_ref[...] = jnp.maximum(cem_ref[...], cem)
        gl_ref[0, 0] += gl_part
        sl_ref[0, 0] += sl_part

    @pl.when(i == GB - 1)
    def _fin():
        gl_ref[0, 0] = gl_ref[0, 0] / (N * D)
        sl_ref[0, 0] = sl_ref[0, 0] / N
        upd = acc_ref[...] * (1.0 / cem_ref[...]) + kt_ref[...]
        nrm = jnp.sqrt(jnp.sum(upd * upd, axis=0, keepdims=True))
        um_ref[...] = (upd / jnp.maximum(nrm, 1e-12)).T


@functools.partial(jax.jit, static_argnames=())
def kernel(query, keys):
    f32 = jnp.float32
    qv = query.reshape(BS, D, HW)

    qr3 = pl.pallas_call(
        _norm_body,
        grid=(BS,),
        in_specs=[pl.BlockSpec((1, D, HW), lambda b: (b, 0, 0))],
        out_specs=pl.BlockSpec((1, HW, D), lambda b: (b, 0, 0)),
        out_shape=jax.ShapeDtypeStruct((BS, HW, D), f32),
    )(qv)
    qr = qr3.reshape(N, D)

    kt, k2c, b, ebc = pl.pallas_call(
        _keys_body,
        grid=(1,),
        in_specs=[pl.BlockSpec((M, D), lambda _: (0, 0))],
        out_specs=[
            pl.BlockSpec((D, M), lambda _: (0, 0)),
            pl.BlockSpec((1, M), lambda _: (0, 0)),
            pl.BlockSpec((1, M), lambda _: (0, 0)),
            pl.BlockSpec((1, M), lambda _: (0, 0)),
        ],
        out_shape=[
            jax.ShapeDtypeStruct((D, M), f32),
            jax.ShapeDtypeStruct((1, M), f32),
            jax.ShapeDtypeStruct((1, M), f32),
            jax.ShapeDtypeStruct((1, M), f32),
        ],
    )(keys)

    sm, cm, csum = pl.pallas_call(
        _passA_body,
        grid=(GA,),
        in_specs=[
            pl.BlockSpec((RA, D), lambda i: (i, 0)),
            pl.BlockSpec((D, M), lambda i: (0, 0)),
            pl.BlockSpec((M, D), lambda i: (0, 0)),
            pl.BlockSpec((1, M), lambda i: (0, 0)),
            pl.BlockSpec((1, M), lambda i: (0, 0)),
        ],
        out_specs=[
            pl.BlockSpec((RA, M), lambda i: (i, 0)),
            pl.BlockSpec((RA, D), lambda i: (i, 0)),
            pl.BlockSpec((1, M), lambda i: (0, 0)),
        ],
        out_shape=[
            jax.ShapeDtypeStruct((N, M), f32),
            jax.ShapeDtypeStruct((N, D), f32),
            jax.ShapeDtypeStruct((1, M), f32),
        ],
    )(qr, kt, keys, b, ebc)

    sub = HW // RB
    sq, uq, um, gl, sl = pl.pallas_call(
        _passB_body,
        grid=(GB,),
        in_specs=[
            pl.BlockSpec((RB, D), lambda i: (i, 0)),
            pl.BlockSpec((RB, D), lambda i: (i, 0)),
            pl.BlockSpec((D, M), lambda i: (0, 0)),
            pl.BlockSpec((1, M), lambda i: (0, 0)),
            pl.BlockSpec((1, M), lambda i: (0, 0)),
            pl.BlockSpec((1, M), lambda i: (0, 0)),
        ],
        out_specs=[
            pl.BlockSpec((RB, M), lambda i: (i, 0)),
            pl.BlockSpec((1, 2 * D, RB), lambda i: (i // sub, 0, i % sub)),
            pl.BlockSpec((M, D), lambda i: (0, 0)),
            pl.BlockSpec(memory_space=pltpu.SMEM),
            pl.BlockSpec(memory_space=pltpu.SMEM),
        ],
        out_shape=[
            jax.ShapeDtypeStruct((N, M), f32),
            jax.ShapeDtypeStruct((BS, 2 * D, HW), f32),
            jax.ShapeDtypeStruct((M, D), f32),
            jax.ShapeDtypeStruct((1, 1), f32),
            jax.ShapeDtypeStruct((1, 1), f32),
        ],
        scratch_shapes=[pltpu.VMEM((D, M), f32), pltpu.VMEM((1, M), f32)],
    )(qr, cm, kt, k2c, b, csum)

    updated_query = uq.reshape(BS, 2 * D, 32, 32)
    return (updated_query, um, sq, sm, gl[0, 0], sl[0, 0])


# speedup vs baseline: 30.4567x; 1.0002x over previous
import functools

import jax
import jax.numpy as jnp
from jax.experimental import pallas as pl
from jax.experimental.pallas import tpu as pltpu

N = 8192
M = 10000
D = 64
RA = 128
GA = N // RA
RB = 256
GB = N // RB
BS = 8
HW = 1024


def _norm_body(q_ref, qr_ref):
    x = q_ref[0]
    ss = jnp.sum(x * x, axis=0, keepdims=True)
    inv = 1.0 / jnp.maximum(jnp.sqrt(ss), 1e-12)
    qr_ref[0] = (x * inv).T


def _keys_body(keys_ref, kt_ref, k2_ref, b_ref, ebc_ref):
    k = keys_ref[...]
    kt = k.T
    kt_ref[...] = kt
    k2 = jnp.sum(kt * kt, axis=0, keepdims=True)
    k2_ref[...] = k2
    b = jnp.sqrt(k2)
    b_ref[...] = b
    ebc_ref[...] = jnp.exp(jnp.max(b) - b)


def _passA_body(qr_ref, kt_ref, keys_ref, b_ref, ebc_ref,
                sm_ref, cm_ref, csum_ref):
    i = pl.program_id(0)
    q = qr_ref[...]
    s = jnp.dot(q, kt_ref[...], preferred_element_type=jnp.float32)

    c = jnp.max(b_ref[...])
    p = jnp.exp(s - c)
    rsum = jnp.sum(p, axis=1, keepdims=True)
    smv = p * (1.0 / rsum)
    sm_ref[...] = smv
    cm_ref[...] = jnp.dot(smv, keys_ref[...], preferred_element_type=jnp.float32)

    ones = jnp.ones((1, RA), jnp.float32)
    colp = jnp.dot(ones, p, preferred_element_type=jnp.float32)
    contrib = colp * ebc_ref[...]

    @pl.when(i == 0)
    def _init():
        csum_ref[...] = contrib

    @pl.when(i > 0)
    def _acc():
        csum_ref[...] += contrib


def _passB_body(qr_ref, cm_ref, kt_ref, k2_ref, b_ref, csum_ref,
                sq_ref, uq_ref, um_ref, gl_ref, sl_ref, acc_ref, cem_ref):
    i = pl.program_id(0)
    q = qr_ref[...]
    s = jnp.dot(q, kt_ref[...], preferred_element_type=jnp.float32)
    e = jnp.exp(s - b_ref[...])
    sq_ref[...] = e * (1.0 / csum_ref[...])

    ninf = jnp.float32(-jnp.inf)
    m1 = jnp.max(s, axis=1, keepdims=True)
    eq = s == m1
    mw = jnp.where(eq, e, 0.0)
    qt = q.T
    contrib = jnp.dot(qt, mw, preferred_element_type=jnp.float32)
    cem = jnp.max(e, axis=0, keepdims=True)

    m2 = jnp.max(jnp.where(eq, ninf, s), axis=1, keepdims=True)
    eq2 = s == m2
    k2 = k2_ref[...]
    k2g = jnp.max(jnp.where(eq, k2, ninf), axis=1, keepdims=True)
    k2n = jnp.max(jnp.where(eq2, k2, ninf), axis=1, keepdims=True)

    q2 = jnp.sum(q * q, axis=1, keepdims=True)
    glr = q2 - 2.0 * m1 + k2g
    dp = jnp.sqrt(glr)
    dn = jnp.sqrt(q2 - 2.0 * m2 + k2n)
    gl_part = jnp.sum(glr)
    sl_part = jnp.sum(jnp.maximum(dp - dn + 1.0, 0.0))

    uq_ref[0, 0:D, :] = qt
    uq_ref[0, D:2 * D, :] = cm_ref[...].T

    @pl.when(i == 0)
    def _init():
        acc_ref[...] = contrib
        cem_ref[...] = cem
        gl_ref[0, 0] = gl_part
        sl_ref[0, 0] = sl_part

    @pl.when(i > 0)
    def _acc():
        acc_ref[...] += contrib
        cem_ref[...] = jnp.maximum(cem_ref[...], cem)
        gl_ref[0, 0] += gl_part
        sl_ref[0, 0] += sl_part

    @pl.when(i == GB - 1)
    def _fin():
        gl_ref[0, 0] = gl_ref[0, 0] / (N * D)
        sl_ref[0, 0] = sl_ref[0, 0] / N
        upd = acc_ref[...] * (1.0 / cem_ref[...]) + kt_ref[...]
        nrm = jnp.sqrt(jnp.sum(upd * upd, axis=0, keepdims=True))
        um_ref[...] = (upd / jnp.maximum(nrm, 1e-12)).T


@functools.partial(jax.jit, static_argnames=())
def kernel(query, keys):
    f32 = jnp.float32
    qv = query.reshape(BS, D, HW)

    qr3 = pl.pallas_call(
        _norm_body,
        grid=(BS,),
        in_specs=[pl.BlockSpec((1, D, HW), lambda b: (b, 0, 0))],
        out_specs=pl.BlockSpec((1, HW, D), lambda b: (b, 0, 0)),
        out_shape=jax.ShapeDtypeStruct((BS, HW, D), f32),
    )(qv)
    qr = qr3.reshape(N, D)

    kt, k2v, b, ebc = pl.pallas_call(
        _keys_body,
        grid=(1,),
        in_specs=[pl.BlockSpec((M, D), lambda _: (0, 0))],
        out_specs=[
            pl.BlockSpec((D, M), lambda _: (0, 0)),
            pl.BlockSpec((1, M), lambda _: (0, 0)),
            pl.BlockSpec((1, M), lambda _: (0, 0)),
            pl.BlockSpec((1, M), lambda _: (0, 0)),
        ],
        out_shape=[
            jax.ShapeDtypeStruct((D, M), f32),
            jax.ShapeDtypeStruct((1, M), f32),
            jax.ShapeDtypeStruct((1, M), f32),
            jax.ShapeDtypeStruct((1, M), f32),
        ],
    )(keys)

    sm, cm, csum = pl.pallas_call(
        _passA_body,
        grid=(GA,),
        in_specs=[
            pl.BlockSpec((RA, D), lambda i: (i, 0)),
            pl.BlockSpec((D, M), lambda i: (0, 0)),
            pl.BlockSpec((M, D), lambda i: (0, 0)),
            pl.BlockSpec((1, M), lambda i: (0, 0)),
            pl.BlockSpec((1, M), lambda i: (0, 0)),
        ],
        out_specs=[
            pl.BlockSpec((RA, M), lambda i: (i, 0)),
            pl.BlockSpec((RA, D), lambda i: (i, 0)),
            pl.BlockSpec((1, M), lambda i: (0, 0)),
        ],
        out_shape=[
            jax.ShapeDtypeStruct((N, M), f32),
            jax.ShapeDtypeStruct((N, D), f32),
            jax.ShapeDtypeStruct((1, M), f32),
        ],
    )(qr, kt, keys, b, ebc)

    sub = HW // RB
    sq, uq, um, gl, sl = pl.pallas_call(
        _passB_body,
        grid=(GB,),
        in_specs=[
            pl.BlockSpec((RB, D), lambda i: (i, 0)),
            pl.BlockSpec((RB, D), lambda i: (i, 0)),
            pl.BlockSpec((D, M), lambda i: (0, 0)),
            pl.BlockSpec((1, M), lambda i: (0, 0)),
            pl.BlockSpec((1, M), lambda i: (0, 0)),
            pl.BlockSpec((1, M), lambda i: (0, 0)),
        ],
        out_specs=[
            pl.BlockSpec((RB, M), lambda i: (i, 0)),
            pl.BlockSpec((1, 2 * D, RB), lambda i: (i // sub, 0, i % sub)),
            pl.BlockSpec((M, D), lambda i: (0, 0)),
            pl.BlockSpec(memory_space=pltpu.SMEM),
            pl.BlockSpec(memory_space=pltpu.SMEM),
        ],
        out_shape=[
            jax.ShapeDtypeStruct((N, M), f32),
            jax.ShapeDtypeStruct((BS, 2 * D, HW), f32),
            jax.ShapeDtypeStruct((M, D), f32),
            jax.ShapeDtypeStruct((1, 1), f32),
            jax.ShapeDtypeStruct((1, 1), f32),
        ],
        scratch_shapes=[pltpu.VMEM((D, M), f32), pltpu.VMEM((1, M), f32)],
    )(qr, cm, kt, k2v, b, csum)

    updated_query = uq.reshape(BS, 2 * D, 32, 32)
    return (updated_query, um, sq, sm, gl[0, 0], sl[0, 0])
